# baseline (speedup 1.0000x reference)
import functools

import jax
import jax.numpy as jnp
from jax import lax
from jax.experimental import pallas as pl
from jax.experimental.pallas import tpu as pltpu
from jax.experimental.pallas import tpu_sc as plsc

D = 64
SEQ = 200
L = 16
NV = D // L
NTW = 4
NBW = 8
TPW = SEQ // NTW
CW = 4096 // NBW


def _hsum16(v):
    lanes = lax.iota(jnp.int32, L)
    dnums = lax.GatherDimensionNumbers(
        offset_dims=(), collapsed_slice_dims=(0,), start_index_map=(0,))
    for sh in (8, 4, 2, 1):
        perm = lax.bitwise_xor(lanes, jnp.int32(sh))
        v = v + lax.gather(v, perm[:, None], dnums, slice_sizes=(1,),
                           mode=lax.GatherScatterMode.PROMISE_IN_BOUNDS)
    return v


def _rsqrt16(v):
    i = lax.bitcast_convert_type(v, jnp.int32)
    i = jnp.int32(0x5F3759DF) - lax.shift_right_logical(i, 1)
    y = lax.bitcast_convert_type(i, jnp.float32)
    for _ in range(2):
        y = y * (1.5 - 0.5 * v * y * y)
    return y


@functools.lru_cache(maxsize=None)
def _make_sc_embed(batch):
    mesh = plsc.VectorSubcoreMesh(core_axis_name="c", subcore_axis_name="s")

    @functools.partial(
        pl.kernel,
        out_type=jax.ShapeDtypeStruct((SEQ, D, batch), jnp.float32),
        mesh=mesh,
        compiler_params=pltpu.CompilerParams(
            use_tc_tiling_on_sc=False, needs_layout_passes=False),
        scratch_types=[
            pltpu.VMEM((CW,), jnp.int32),
            pltpu.VMEM((CW,), jnp.int32),
            pltpu.VMEM((D, SEQ), jnp.float32),
            pltpu.VMEM((D,), jnp.float32),
            pltpu.VMEM((D,), jnp.float32),
            pltpu.VMEM((CW, D), jnp.float32),
            pltpu.VMEM((CW, D), jnp.float32),
            pltpu.VMEM((D, CW), jnp.float32),
            pltpu.SemaphoreType.DMA,
            pltpu.SemaphoreType.DMA,
            pltpu.SemaphoreType.DMA,
            pltpu.SemaphoreType.DMA,
        ],
    )
    def sc_embed(idt_hbm, wt_hbm, post_hbm, sc_hbm, sh_hbm, out_hbm,
                 ids0_v, ids1_v, post_v, scale_v, shift_v, rows0_v, rows1_v,
                 outb_v, isem0, isem1, gsem0, gsem1):
        wid = lax.axis_index("s") * 2 + lax.axis_index("c")
        t0 = (wid // NBW) * TPW
        bb = pl.multiple_of((wid % NBW) * CW, CW)
        pltpu.sync_copy(post_hbm.at[:, pl.ds(0, SEQ)], post_v)
        pltpu.sync_copy(sc_hbm, scale_v)
        pltpu.sync_copy(sh_hbm, shift_v)

        scs = [scale_v[pl.ds(q * L, L)] for q in range(NV)]
        shs = [shift_v[pl.ds(q * L, L)] for q in range(NV)]
        ids = (ids0_v, ids1_v)
        isems = (isem0, isem1)
        gsems = (gsem0, gsem1)
        rows = (rows0_v, rows1_v)

        def start_ids(tt, b):
            pltpu.async_copy(
                idt_hbm.at[t0 + tt, pl.ds(bb, CW)], ids[b], isems[b])

        def drain_ids(b):
            pltpu.make_async_copy(
                idt_hbm.at[0, pl.ds(bb, CW)], ids[b], isems[b]).wait()

        def start_gather(tt, b):
            pltpu.async_copy(wt_hbm.at[ids[b]], rows[b], gsems[b])

        def drain_gather(b):
            pltpu.make_async_copy(
                wt_hbm.at[pl.ds(0, CW)], rows[b], gsems[b]).wait()

        def compute(tt, b):
            tv = lax.broadcast(t0 + tt, (L,))
            lanes = lax.iota(jnp.int32, L)
            pe = [plsc.load_gather(post_v, [q * L + lanes, tv])
                  for q in range(NV)]

            @plsc.parallel_loop(0, CW, step=1, unroll=8)
            def _row(j):
                e = [rows[b][j, pl.ds(q * L, L)] + pe[q]
                     for q in range(NV)]
                s = (e[0] + e[1]) + (e[2] + e[3])
                q2 = ((e[0] * e[0] + e[1] * e[1])
                      + (e[2] * e[2] + e[3] * e[3]))
                mu = _hsum16(s) * (1.0 / D)
                exx = _hsum16(q2) * (1.0 / D)
                y = _rsqrt16(exx - mu * mu + 1e-12)
                jv = lax.broadcast(j, (L,))
                ln = lax.iota(jnp.int32, L)
                for q in range(NV):
                    plsc.store_scatter(
                        outb_v, [q * L + ln, jv],
                        ((e[q] - mu) * y) * scs[q] + shs[q])

        def body(tt, b, first, last, penult=False):
            drain_gather(b)
            if not last:
                if not penult:
                    start_ids(tt + 2, b)
                drain_ids(1 - b)
                start_gather(tt + 1, 1 - b)
            compute(tt, b)
            pltpu.sync_copy(outb_v, out_hbm.at[t0 + tt, :, pl.ds(bb, CW)])

        start_ids(0, 0)
        drain_ids(0)
        start_gather(0, 0)
        start_ids(1, 1)
        body(0, 0, True, False)
        body(1, 1, True, False)

        def main(s, carry):
            tt = 2 * s
            body(tt, 0, False, False)
            body(tt + 1, 1, False, False)
            return carry

        lax.fori_loop(1, TPW // 2 - 1, main, 0)
        body(TPW - 2, 0, False, False, penult=True)
        body(TPW - 1, 1, False, True)

    return sc_embed


def kernel(input_ids, word_table, pos_table, scale, shift):
    B, S = input_ids.shape
    assert S == SEQ and word_table.shape[1] == D
    idt = input_ids.T.astype(jnp.int32)
    post = pos_table.T
    out = _make_sc_embed(B)(idt, word_table, post, scale, shift)
    return jnp.transpose(out, (2, 0, 1))


# speedup vs baseline: 1.0904x; 1.0904x over previous
import functools

import jax
import jax.numpy as jnp
from jax import lax
from jax.experimental import pallas as pl
from jax.experimental.pallas import tpu as pltpu
from jax.experimental.pallas import tpu_sc as plsc

D = 64
SEQ = 200
L = 16
NV = D // L
NTW = 4
NBW = 8
TPW = SEQ // NTW
CW = 4096 // NBW
CH = CW // 2
NU = TPW * 2


def _hsum16(v):
    lanes = lax.iota(jnp.int32, L)
    dnums = lax.GatherDimensionNumbers(
        offset_dims=(), collapsed_slice_dims=(0,), start_index_map=(0,))
    for sh in (8, 4, 2, 1):
        perm = lax.bitwise_xor(lanes, jnp.int32(sh))
        v = v + lax.gather(v, perm[:, None], dnums, slice_sizes=(1,),
                           mode=lax.GatherScatterMode.PROMISE_IN_BOUNDS)
    return v


def _rsqrt16(v):
    i = lax.bitcast_convert_type(v, jnp.int32)
    i = jnp.int32(0x5F3759DF) - lax.shift_right_logical(i, 1)
    y = lax.bitcast_convert_type(i, jnp.float32)
    for _ in range(2):
        y = y * (1.5 - 0.5 * v * y * y)
    return y


@functools.lru_cache(maxsize=None)
def _make_sc_embed(batch):
    mesh = plsc.VectorSubcoreMesh(core_axis_name="c", subcore_axis_name="s")

    @functools.partial(
        pl.kernel,
        out_type=jax.ShapeDtypeStruct((SEQ, D, batch), jnp.float32),
        mesh=mesh,
        compiler_params=pltpu.CompilerParams(
            use_tc_tiling_on_sc=False, needs_layout_passes=False),
        scratch_types=[
            pltpu.VMEM((CH,), jnp.int32),
            pltpu.VMEM((CH,), jnp.int32),
            pltpu.VMEM((D, SEQ), jnp.float32),
            pltpu.VMEM((D,), jnp.float32),
            pltpu.VMEM((D,), jnp.float32),
            pltpu.VMEM((CH, D), jnp.float32),
            pltpu.VMEM((CH, D), jnp.float32),
            pltpu.VMEM((D, CW), jnp.float32),
            pltpu.VMEM((D, CW), jnp.float32),
            pltpu.SemaphoreType.DMA,
            pltpu.SemaphoreType.DMA,
            pltpu.SemaphoreType.DMA,
            pltpu.SemaphoreType.DMA,
            pltpu.SemaphoreType.DMA,
            pltpu.SemaphoreType.DMA,
        ],
    )
    def sc_embed(idt_hbm, wt_hbm, post_hbm, sc_hbm, sh_hbm, out_hbm,
                 ids0_v, ids1_v, post_v, scale_v, shift_v, rows0_v, rows1_v,
                 outb0_v, outb1_v, isem0, isem1, gsem0, gsem1, osem0, osem1):
        wid = lax.axis_index("s") * 2 + lax.axis_index("c")
        t0 = (wid // NBW) * TPW
        bb = pl.multiple_of((wid % NBW) * CW, CW)
        pltpu.sync_copy(post_hbm.at[:, pl.ds(0, SEQ)], post_v)
        pltpu.sync_copy(sc_hbm, scale_v)
        pltpu.sync_copy(sh_hbm, shift_v)

        scs = [scale_v[pl.ds(q * L, L)] for q in range(NV)]
        shs = [shift_v[pl.ds(q * L, L)] for q in range(NV)]
        ids = (ids0_v, ids1_v)
        isems = (isem0, isem1)
        gsems = (gsem0, gsem1)
        osems = (osem0, osem1)
        rows = (rows0_v, rows1_v)
        outbs = (outb0_v, outb1_v)

        def start_ids(tt, h, b):
            pltpu.async_copy(
                idt_hbm.at[t0 + tt, pl.ds(bb + h * CH, CH)], ids[b],
                isems[b])

        def drain_ids(b):
            pltpu.make_async_copy(
                idt_hbm.at[0, pl.ds(bb, CH)], ids[b], isems[b]).wait()

        def start_gather(b):
            pltpu.async_copy(wt_hbm.at[ids[b]], rows[b], gsems[b])

        def drain_gather(b):
            pltpu.make_async_copy(
                wt_hbm.at[pl.ds(0, CH)], rows[b], gsems[b]).wait()

        def start_out(tt, ob):
            pltpu.async_copy(
                outbs[ob], out_hbm.at[t0 + tt, :, pl.ds(bb, CW)], osems[ob])

        def drain_out(ob):
            pltpu.make_async_copy(
                out_hbm.at[0, :, pl.ds(bb, CW)], outbs[ob], osems[ob]).wait()

        def compute(tt, h, rb, ob):
            tv = lax.broadcast(t0 + tt, (L,))
            lanes = lax.iota(jnp.int32, L)
            pe = [plsc.load_gather(post_v, [q * L + lanes, tv])
                  for q in range(NV)]

            @plsc.parallel_loop(0, CH, step=1, unroll=4)
            def _row(j):
                e = [rows[rb][j, pl.ds(q * L, L)] + pe[q]
                     for q in range(NV)]
                s = (e[0] + e[1]) + (e[2] + e[3])
                q2 = ((e[0] * e[0] + e[1] * e[1])
                      + (e[2] * e[2] + e[3] * e[3]))
                mu = _hsum16(s) * (1.0 / D)
                exx = _hsum16(q2) * (1.0 / D)
                y = _rsqrt16(exx - mu * mu + 1e-12)
                jv = lax.broadcast(j, (L,))
                ln = lax.iota(jnp.int32, L)
                for q in range(NV):
                    plsc.store_scatter(
                        outbs[ob], [q * L + ln, jv + h * CH],
                        ((e[q] - mu) * y) * scs[q] + shs[q])

        def body(tt, k, ids2=True, gat1=True, drn=True):
            h = k % 2
            rb = k % 2
            ob = (k // 2) % 2
            drain_gather(rb)
            if ids2:
                start_ids(tt + 1, h, rb)
            if gat1:
                drain_ids(1 - rb)
                start_gather(1 - rb)
            if h == 0 and drn:
                drain_out(ob)
            compute(tt, h, rb, ob)
            if h == 1:
                start_out(tt, ob)

        start_ids(0, 0, 0)
        drain_ids(0)
        start_gather(0)
        start_ids(0, 1, 1)
        for k in range(4):
            body(k // 2, k, drn=False)

        def main(s, carry):
            for k in range(4):
                body(2 * s + k // 2, k)
            return carry

        lax.fori_loop(1, NU // 4 - 1, main, 0)
        for k in range(4):
            u = NU - 4 + k
            body(TPW - 2 + k // 2, k,
                 ids2=(u + 2 < NU), gat1=(u + 1 < NU))
        drain_out(0)
        drain_out(1)

    return sc_embed


def kernel(input_ids, word_table, pos_table, scale, shift):
    B, S = input_ids.shape
    assert S == SEQ and word_table.shape[1] == D
    idt = input_ids.T.astype(jnp.int32)
    post = pos_table.T
    out = _make_sc_embed(B)(idt, word_table, post, scale, shift)
    return jnp.transpose(out, (2, 0, 1))


# speedup vs baseline: 1.3647x; 1.2516x over previous
import functools

import jax
import jax.numpy as jnp
from jax import lax
from jax.experimental import pallas as pl
from jax.experimental.pallas import tpu as pltpu
from jax.experimental.pallas import tpu_sc as plsc

D = 64
SEQ = 200
L = 16
NV = D // L
NC = 2
NS = 16
NW = NC * NS
NB = 3


def _hsum16(v):
    lanes = lax.iota(jnp.int32, L)
    dnums = lax.GatherDimensionNumbers(
        offset_dims=(), collapsed_slice_dims=(0,), start_index_map=(0,))
    for sh in (8, 4, 2, 1):
        perm = lax.bitwise_xor(lanes, jnp.int32(sh))
        v = v + lax.gather(v, perm[:, None], dnums, slice_sizes=(1,),
                           mode=lax.GatherScatterMode.PROMISE_IN_BOUNDS)
    return v


def _rsqrt16(v):
    i = lax.bitcast_convert_type(v, jnp.int32)
    i = jnp.int32(0x5F3759DF) - lax.shift_right_logical(i, 1)
    y = lax.bitcast_convert_type(i, jnp.float32)
    for _ in range(2):
        y = y * (1.5 - 0.5 * v * y * y)
    return y


@functools.lru_cache(maxsize=None)
def _make_sc_embed(n_rows):
    rows_per_w = n_rows // NW
    ch_per_w = rows_per_w // SEQ
    mesh = plsc.VectorSubcoreMesh(core_axis_name="c", subcore_axis_name="s")

    @functools.partial(
        pl.kernel,
        out_type=jax.ShapeDtypeStruct((n_rows, D), jnp.float32),
        mesh=mesh,
        compiler_params=pltpu.CompilerParams(use_tc_tiling_on_sc=False),
        scratch_types=(
            [pltpu.VMEM((rows_per_w,), jnp.int32)]
            + [pltpu.VMEM((SEQ, D), jnp.float32)]
            + [pltpu.VMEM((D,), jnp.float32)] * 2
            + [pltpu.VMEM((SEQ, D), jnp.float32)] * NB
            + [pltpu.SemaphoreType.DMA] * (2 * NB)
        ),
    )
    def sc_embed(idx_hbm, wt_hbm, pos_hbm, sc_hbm, sh_hbm, out_hbm,
                 idx_v, pos_v, scale_v, shift_v, r0, r1, r2,
                 g0, g1, g2, o0, o1, o2):
        wid = lax.axis_index("s") * NC + lax.axis_index("c")
        row0 = wid * rows_per_w
        pltpu.sync_copy(idx_hbm.at[pl.ds(row0, rows_per_w)], idx_v)
        pltpu.sync_copy(pos_hbm.at[pl.ds(0, SEQ)], pos_v)
        pltpu.sync_copy(sc_hbm, scale_v)
        pltpu.sync_copy(sh_hbm, shift_v)

        scs = [scale_v[pl.ds(q * L, L)] for q in range(NV)]
        shs = [shift_v[pl.ds(q * L, L)] for q in range(NV)]
        rows = (r0, r1, r2)
        gsems = (g0, g1, g2)
        osems = (o0, o1, o2)

        def start_gather(t, b):
            pltpu.async_copy(
                wt_hbm.at[idx_v.at[pl.ds(t * SEQ, SEQ)]], rows[b], gsems[b])

        def drain_gather(b):
            pltpu.make_async_copy(
                wt_hbm.at[pl.ds(0, SEQ)], rows[b], gsems[b]).wait()

        def start_out(t, b):
            pltpu.async_copy(
                rows[b], out_hbm.at[pl.ds(row0 + t * SEQ, SEQ)], osems[b])

        def drain_out(b):
            pltpu.make_async_copy(
                out_hbm.at[pl.ds(0, SEQ)], rows[b], osems[b]).wait()

        def compute(b):
            @plsc.parallel_loop(0, SEQ, step=1, unroll=8)
            def _row(j):
                e = [rows[b][j, pl.ds(q * L, L)] + pos_v[j, pl.ds(q * L, L)]
                     for q in range(NV)]
                s = (e[0] + e[1]) + (e[2] + e[3])
                q2 = ((e[0] * e[0] + e[1] * e[1])
                      + (e[2] * e[2] + e[3] * e[3]))
                mu = _hsum16(s) * (1.0 / D)
                exx = _hsum16(q2) * (1.0 / D)
                y = _rsqrt16(exx - mu * mu + 1e-12)
                for q in range(NV):
                    rows[b][j, pl.ds(q * L, L)] = (
                        ((e[q] - mu) * y) * scs[q] + shs[q])

        def body(t, b, drn=True, nxt=True):
            drain_gather(b)
            if drn:
                drain_out((b + 1) % NB)
            if nxt:
                start_gather(t + 1, (b + 1) % NB)
            compute(b)
            start_out(t, b)

        start_gather(0, 0)
        body(0, 0, drn=False)
        body(1, 1, drn=False)

        def main(s, carry):
            t = 3 * s + 2
            for k in range(3):
                body(t + k, (2 + k) % NB)
            return carry

        n_main = (ch_per_w - 2 - 3) // 3
        lax.fori_loop(0, n_main, main, 0)
        for t in range(3 * n_main + 2, ch_per_w):
            body(t, t % NB, nxt=(t + 1 < ch_per_w))
        drain_out((ch_per_w - 2) % NB)
        drain_out((ch_per_w - 1) % NB)

    return sc_embed


def kernel(input_ids, word_table, pos_table, scale, shift):
    B, S = input_ids.shape
    assert S == SEQ and word_table.shape[1] == D
    idx = input_ids.reshape(-1).astype(jnp.int32)
    out = _make_sc_embed(B * S)(idx, word_table, pos_table, scale, shift)
    return out.reshape(B, S, D)
